# baseline (speedup 1.0000x reference)
import functools

import jax
import jax.numpy as jnp
from jax import lax
from jax.experimental import pallas as pl
from jax.experimental.pallas import tpu as pltpu
from jax.experimental.pallas import tpu_sc as plsc

B = 16384
D = 64
C = 5
VOC = 1000000
NW = 32
W = 512
NBLK = VOC // W + 1
NBLKP = 1984
N_T = B
N_C = B * C
PAD = 128

_I16 = lambda x: jnp.full((16,), x, jnp.int32)


def _iota16():
    return jnp.arange(16, dtype=jnp.int32)


def _hist_body(tidx, cidx, hw_t, hw_c, sbuf, hg0, hg1, hg2, hg3, hsum):
    hgs = [hg0, hg1, hg2, hg3]
    wid = lax.axis_index("s") * 2 + lax.axis_index("c")
    ones = _I16(1)
    zeros16 = jnp.zeros((16,), jnp.int32)

    for idx_hbm, n, hw_out in ((tidx, N_T, hw_t), (cidx, N_C, hw_c)):
        sl = n // NW
        my_lo = wid * sl

        def zk(k, _):
            k16 = pl.multiple_of(16 * k, 16)
            for h in hgs:
                h[pl.ds(k16, 16)] = zeros16
            return 0

        lax.fori_loop(0, NBLKP // 16, zk, 0)
        pltpu.sync_copy(idx_hbm.at[pl.ds(my_lo, sl)], sbuf.at[pl.ds(0, sl)])

        def hj(j, _):
            for u in range(4):
                o = pl.multiple_of(64 * j + 16 * u, 16)
                v = sbuf[pl.ds(o, 16)]
                plsc.addupdate_scatter(hgs[u], [v >> 9], ones)
            return 0

        lax.fori_loop(0, sl // 64, hj, 0)

        def sk(k, _):
            k16 = pl.multiple_of(16 * k, 16)
            hsum[pl.ds(k16, 16)] = (
                hgs[0][pl.ds(k16, 16)] + hgs[1][pl.ds(k16, 16)]
                + hgs[2][pl.ds(k16, 16)] + hgs[3][pl.ds(k16, 16)])
            return 0

        lax.fori_loop(0, NBLKP // 16, sk, 0)
        pltpu.sync_copy(hsum, hw_out.at[wid])


def _bin_body(tidx, cidx, hw_t, hw_c, pv_t, pp_t, pv_c, pp_c, off_t, off_c,
              sbuf, hw_v, off_v, base_v, slot2, pos2, sem):
    wid = lax.axis_index("s") * 2 + lax.axis_index("c")
    iota = _iota16()
    ones = _I16(1)
    zeros16 = jnp.zeros((16,), jnp.int32)
    masks = [iota == l for l in range(16)]

    for idx_hbm, n, hw_in, pv_out, pp_out, off_out in (
            (tidx, N_T, hw_t, pv_t, pp_t, off_t),
            (cidx, N_C, hw_c, pv_c, pp_c, off_c)):
        sl = n // NW
        my_lo = wid * sl

        pltpu.sync_copy(hw_in, hw_v)

        lane15 = iota == 15

        def pfx(k, carry):
            k16 = pl.multiple_of(16 * k, 16)
            h = zeros16
            he = zeros16
            for u in range(NW):
                row = hw_v[u, pl.ds(k16, 16)]
                h = h + row
                he = he + jnp.where(_I16(u) < _I16(wid), row, zeros16)
            cs = jnp.cumsum(h)
            off_v[pl.ds(k16, 16)] = cs - h + carry
            base_v[pl.ds(k16, 16)] = cs - h + carry + he
            return carry + jnp.sum(jnp.where(lane15, cs, 0))

        lax.fori_loop(0, NBLKP // 16, pfx, jnp.int32(0))

        @pl.when(wid == 0)
        def _():
            pltpu.sync_copy(off_v, off_out)

        pltpu.sync_copy(idx_hbm.at[pl.ds(my_lo, sl)], sbuf.at[pl.ds(0, sl)])

        def slotch(j, _):
            v = sbuf[pl.ds(16 * j, 16)]
            blk = v >> 9
            slot = zeros16
            for l in range(16):
                g = plsc.load_gather(base_v, [blk], mask=masks[l])
                slot = jnp.where(masks[l], g, slot)
                plsc.addupdate_scatter(base_v, [blk], ones, mask=masks[l])
            r, q = j // 8, j % 8
            slot2[r, pl.ds(16 * q, 16)] = slot
            pos2[r, pl.ds(16 * q, 16)] = _I16(my_lo + 16 * j) + iota
            return 0

        lax.fori_loop(0, sl // 16, slotch, 0)

        cps = []
        for ch in range(sl // 128):
            cps.append(pltpu.async_copy(
                sbuf.at[pl.ds(ch * 128, 128)],
                pv_out.at[slot2.at[ch]], sem))
            cps.append(pltpu.async_copy(
                pos2.at[ch], pp_out.at[slot2.at[ch]], sem))
        for cp in cps:
            cp.wait()


def _sweep_body(ttab_t, ctab_t, pv_t, pp_t, pv_c, pp_c, off_t, off_c,
                gat_t, gat_c, buf0, buf1, tailb, stage0, stage1,
                off_v, vb0, vb1, pb0, pb1, sidx0, sidx1,
                sem, sem2a, sem2b, sem3):
    wid = lax.axis_index("s") * 2 + lax.axis_index("c")
    iota = _iota16()
    cnt = jnp.where(wid < 2, 62, jnp.where(wid == 31, 60, 61))
    lo = 61 * wid + jnp.minimum(wid, 2)
    bufs = [buf0, buf1]
    vbs = [vb0, vb1]
    pbs = [pb0, pb1]
    stages = [stage0, stage1]
    sidxs = [sidx0, sidx1]
    sem2s = [sem2a, sem2b]

    def initdump(r, _):
        r16 = pl.multiple_of(16 * r, 16)
        sidx0[pl.ds(r16, 16)] = _I16(N_T + 16 * r) + _iota16()
        sidx1[pl.ds(r16, 16)] = _I16(N_T + 16 * r) + _iota16()
        return 0

    lax.fori_loop(0, 8, initdump, 0)
    pltpu.async_copy(stage0, gat_t.at[sidx0], sem2a)
    pltpu.async_copy(stage1, gat_t.at[sidx1], sem2b)

    for tab, pv_in, pp_in, off_in, gat, ndump in (
            (ttab_t, pv_t, pp_t, off_t, gat_t, N_T),
            (ctab_t, pv_c, pp_c, off_c, gat_c, N_C)):
        pltpu.sync_copy(off_in, off_v)
        dump = ndump

        def blk_dma(j, slot):
            pltpu.async_copy(tab.at[:, pl.ds(j * W, W)], bufs[slot], sem)

        def blk_wait(slot):
            pltpu.make_async_copy(tab.at[:, pl.ds(0, W)],
                                  bufs[slot], sem).wait()

        def bounds(j):
            j8 = j & ~jnp.int32(7)
            ov = off_v[pl.ds(pl.multiple_of(j8, 8), 16)]
            l0 = j - j8
            s0 = jnp.sum(jnp.where(iota == _I16(l0), ov, 0))
            s1 = jnp.sum(jnp.where(iota == _I16(l0 + 1), ov, 0))
            return s0, s1

        def idx_dma(j, slot):
            s0, _ = bounds(j)
            q0 = pl.multiple_of(s0 & ~jnp.int32(7), 8)
            pltpu.async_copy(pv_in.at[pl.ds(q0, 128)], vbs[slot], sem3)
            pltpu.async_copy(pp_in.at[pl.ds(q0, 128)], pbs[slot], sem3)

        def idx_wait(slot):
            pltpu.make_async_copy(pv_in.at[pl.ds(0, 128)],
                                  vbs[slot], sem3).wait()
            pltpu.make_async_copy(pp_in.at[pl.ds(0, 128)],
                                  pbs[slot], sem3).wait()

        def process(j, buf, vbuf, pbuf, stage, sidx, sem2):
            s0, s1 = bounds(j)
            a = s0 & ~jnp.int32(7)
            trips = (s1 - a + 127) >> 7

            def chunk(k, _):
                q0 = pl.multiple_of(a + 128 * k, 8)

                @pl.when(k > 0)
                def _():
                    pltpu.sync_copy(pv_in.at[pl.ds(q0, 128)], vbuf)
                    pltpu.sync_copy(pp_in.at[pl.ds(q0, 128)], pbuf)

                pltpu.make_async_copy(stage, gat.at[sidx], sem2).wait()

                def rbody(r, _):
                    r16 = pl.multiple_of(16 * r, 16)
                    q = _I16(q0 + 16 * r) + iota
                    m = (q >= _I16(s0)) & (q < _I16(s1))
                    v = vbuf[pl.ds(r16, 16)]
                    col = v - _I16(j * W)
                    row16 = _I16(16 * r) + iota
                    for d in range(D):
                        val = plsc.load_gather(
                            buf, [_I16(d), col], mask=m)
                        plsc.store_scatter(
                            stage, [row16, _I16(d)], val, mask=m)
                    pb = pbuf[pl.ds(r16, 16)]
                    spread = _I16(dump) + ((row16 + _I16(4 * wid)) & _I16(127))
                    pbuf[pl.ds(r16, 16)] = jnp.where(m, pb, spread)
                    return 0

                lax.fori_loop(0, 8, rbody, 0)

                def cpb(r, _):
                    r16 = pl.multiple_of(16 * r, 16)
                    sidx[pl.ds(r16, 16)] = pbuf[pl.ds(r16, 16)]
                    return 0

                lax.fori_loop(0, 8, cpb, 0)
                pltpu.async_copy(stage, gat.at[sidx], sem2)
                return 0

            lax.fori_loop(0, trips, chunk, 0)

        blk_dma(lo, 0)
        idx_dma(lo, 0)

        @pl.when(cnt > 1)
        def _():
            blk_dma(lo + 1, 1)
            idx_dma(lo + 1, 1)

        def pair(g, _):
            for b in range(2):
                j = 2 * g + b

                @pl.when(j < cnt)
                def _():
                    blk_wait(b)
                    idx_wait(b)
                    process(lo + j, bufs[b], vbs[b], pbs[b],
                            stages[b], sidxs[b], sem2s[b])

                    @pl.when(j + 2 < cnt)
                    def _():
                        blk_dma(lo + j + 2, b)
                        idx_dma(lo + j + 2, b)
            return 0

        lax.fori_loop(0, 31, pair, 0)

        @pl.when(wid == 31)
        def _():
            pltpu.sync_copy(tab.at[:, pl.ds((NBLK - 1) * W, 64)], tailb)
            idx_dma(jnp.int32(NBLK - 1), 0)
            idx_wait(0)
            process(jnp.int32(NBLK - 1), tailb, vbs[0], pbs[0],
                    stage0, sidx0, sem2a)

    pltpu.make_async_copy(stage0, gat_c.at[sidx0], sem2a).wait()
    pltpu.make_async_copy(stage1, gat_c.at[sidx1], sem2b).wait()


def _dots_body(gat_t, gat_c, out_hbm, trows_v, crows_v, out_v, sem):
    wid = lax.axis_index("s") * 2 + lax.axis_index("c")
    iota = _iota16()
    lane15 = iota == 15

    for chunk in range(4):
        base = (wid * 4 + chunk) * 128
        cp1 = pltpu.async_copy(
            gat_t.at[pl.ds(base, 128)], trows_v, sem)
        cp2 = pltpu.async_copy(
            gat_c.at[pl.ds(base * C, 128 * C)], crows_v, sem)
        cp1.wait()
        cp2.wait()

        def body(g, _):
            for bl in range(16):
                i = g * 16 + bl
                t = [trows_v[i, pl.ds(16 * k, 16)] for k in range(4)]
                for c in range(C):
                    r = i * C + c
                    acc = t[0] * crows_v[r, pl.ds(0, 16)]
                    for k in range(1, 4):
                        acc = acc + t[k] * crows_v[r, pl.ds(16 * k, 16)]
                    cums = jnp.cumsum(acc)
                    plsc.store_scatter(out_v, [_I16(r)], cums, mask=lane15)
            return 0

        lax.fori_loop(0, 8, body, 0)
        pltpu.sync_copy(out_v, out_hbm.at[pl.ds(base * C, 128 * C)])


def kernel(target, context, target_table, context_table):
    mesh = plsc.VectorSubcoreMesh(core_axis_name="c", subcore_axis_name="s")
    ctx_flat = context.reshape(N_C).astype(jnp.int32)
    tgt = target.astype(jnp.int32)

    i32 = jnp.int32
    hists = functools.partial(
        pl.kernel, mesh=mesh,
        compiler_params=pltpu.CompilerParams(
            needs_layout_passes=False, use_tc_tiling_on_sc=False),
        out_type=(
            jax.ShapeDtypeStruct((NW, NBLKP), i32),
            jax.ShapeDtypeStruct((NW, NBLKP), i32),
        ),
        scratch_types=(
            [pltpu.VMEM((N_C // NW,), i32)]
            + [pltpu.VMEM((NBLKP,), i32)] * 5
        ),
    )(_hist_body)

    hw_t, hw_c = hists(tgt, ctx_flat)

    binned = functools.partial(
        pl.kernel, mesh=mesh,
        compiler_params=pltpu.CompilerParams(
            needs_layout_passes=False, use_tc_tiling_on_sc=False),
        out_type=(
            jax.ShapeDtypeStruct((N_T + PAD,), i32),
            jax.ShapeDtypeStruct((N_T + PAD,), i32),
            jax.ShapeDtypeStruct((N_C + PAD,), i32),
            jax.ShapeDtypeStruct((N_C + PAD,), i32),
            jax.ShapeDtypeStruct((NBLKP,), i32),
            jax.ShapeDtypeStruct((NBLKP,), i32),
        ),
        scratch_types=[
            pltpu.VMEM((N_C // NW,), i32),
            pltpu.VMEM((NW, NBLKP), i32),
            pltpu.VMEM((NBLKP,), i32),
            pltpu.VMEM((NBLKP,), i32),
            pltpu.VMEM((N_C // NW // 128, 128), i32),
            pltpu.VMEM((N_C // NW // 128, 128), i32),
            pltpu.SemaphoreType.DMA,
        ],
    )(_bin_body)

    pv_t, pp_t, pv_c, pp_c, off_t, off_c = binned(tgt, ctx_flat, hw_t, hw_c)

    swept = functools.partial(
        pl.kernel, mesh=mesh,
        compiler_params=pltpu.CompilerParams(
            needs_layout_passes=False, use_tc_tiling_on_sc=True),
        out_type=(
            jax.ShapeDtypeStruct((N_T + PAD, 128), jnp.float32),
            jax.ShapeDtypeStruct((N_C + PAD, 128), jnp.float32),
        ),
        scratch_types=[
            pltpu.VMEM((D, W), jnp.float32),
            pltpu.VMEM((D, W), jnp.float32),
            pltpu.VMEM((D, 64), jnp.float32),
            pltpu.VMEM((128, 128), jnp.float32),
            pltpu.VMEM((128, 128), jnp.float32),
            pltpu.VMEM((NBLKP,), i32),
            pltpu.VMEM((128,), i32),
            pltpu.VMEM((128,), i32),
            pltpu.VMEM((128,), i32),
            pltpu.VMEM((128,), i32),
            pltpu.VMEM((128,), i32),
            pltpu.VMEM((128,), i32),
            pltpu.SemaphoreType.DMA,
            pltpu.SemaphoreType.DMA,
            pltpu.SemaphoreType.DMA,
            pltpu.SemaphoreType.DMA,
        ],
    )(_sweep_body)

    gat_t, gat_c = swept(target_table.T, context_table.T,
                         pv_t, pp_t, pv_c, pp_c, off_t, off_c)

    dots = functools.partial(
        pl.kernel, mesh=mesh,
        compiler_params=pltpu.CompilerParams(
            needs_layout_passes=False, use_tc_tiling_on_sc=True),
        out_type=jax.ShapeDtypeStruct((N_C,), jnp.float32),
        scratch_types=[
            pltpu.VMEM((128, 128), jnp.float32),
            pltpu.VMEM((128 * C, 128), jnp.float32),
            pltpu.VMEM((128 * C,), jnp.float32),
            pltpu.SemaphoreType.DMA,
        ],
    )(_dots_body)

    out = dots(gat_t, gat_c)
    return out.reshape(B, C)
